# Initial kernel scaffold; baseline (speedup 1.0000x reference)
import jax
import jax.numpy as jnp
from jax.experimental import pallas as pl


def kernel(x, edge_attr, global_attr, params, edge_index):
    raise NotImplementedError("write your pallas kernel here")


# speedup vs baseline: 1.1871x; 1.1871x over previous
import functools

import jax
import jax.numpy as jnp
import numpy as np
from jax.experimental import pallas as pl

N = 10000
E = 320000
D_NODE = 128
D_EDGE = 16
H = 64

BE = 2560


def _edge1_body(edge_ref, rg_ref, sg_ref, wp_ref, cp_ref, e1_ref, xc_ref):
    rg = rg_ref[...]
    sg = sg_ref[...]
    p = jnp.dot(edge_ref[...], wp_ref[...], preferred_element_type=jnp.float32)
    pre = p + cp_ref[...] + rg[:, :128] + sg[:, :128]
    e1_ref[...] = jnp.maximum(pre, 0.0)
    xc_ref[...] = rg[:, 128:144] + sg[:, 128:144]


def _edge2_body(e1_ref, bgr_ref, bgs_ref, xc_ref, wbd_ref, cd_ref, w2_ref,
                b2_ref, out_ref):
    q = jnp.dot(e1_ref[...], wbd_ref[...], preferred_element_type=jnp.float32)
    h = jnp.maximum(q + cd_ref[...] + bgr_ref[...] + bgs_ref[...], 0.0)
    o = jnp.dot(h, w2_ref[...], preferred_element_type=jnp.float32) + b2_ref[...]
    xc = xc_ref[...]
    out_ref[...] = o[:, 0:1] * (xc[:, 0:1] - o[:, 1:2] * xc[:, 1:2])


def _tables_body(x_ref, wr_ref, ws_ref, wn_ref, r_ref, s_ref, nx_ref):
    x = x_ref[...]
    r_ref[...] = jnp.dot(x, wr_ref[...], preferred_element_type=jnp.float32)
    s_ref[...] = jnp.dot(x, ws_ref[...], preferred_element_type=jnp.float32)
    nx_ref[...] = jnp.dot(x, wn_ref[...], preferred_element_type=jnp.float32)


def _node_body(aggp_ref, nx_ref, u_ref, wagg_ref, cn_ref, gw_e_ref, gw_n_ref,
               gw_u_ref, gb_ref, wdr_ref, wds_ref, wdu_ref, bd_ref,
               br_ref, bs_ref, cd_ref):
    agg = aggp_ref[0] + aggp_ref[1]
    n1 = jnp.maximum(
        jnp.dot(agg, wagg_ref[...], preferred_element_type=jnp.float32)
        + nx_ref[...] + cn_ref[...], 0.0)
    me = jnp.sum(agg, axis=0, keepdims=True) * (1.0 / E)
    mn = jnp.sum(n1, axis=0, keepdims=True) * (1.0 / N)
    u2 = u_ref[...]
    u1 = jnp.maximum(
        jnp.dot(me, gw_e_ref[...], preferred_element_type=jnp.float32)
        + jnp.dot(mn, gw_n_ref[...], preferred_element_type=jnp.float32)
        + jnp.dot(u2, gw_u_ref[...], preferred_element_type=jnp.float32)
        + gb_ref[...], 0.0)
    br_ref[...] = jnp.dot(n1, wdr_ref[...], preferred_element_type=jnp.float32)
    bs_ref[...] = jnp.dot(n1, wds_ref[...], preferred_element_type=jnp.float32)
    cd_ref[...] = jnp.dot(u1, wdu_ref[...], preferred_element_type=jnp.float32) + bd_ref[...]


def _blockdiag(a, b):
    da0, da1 = a.shape
    db0, db1 = b.shape
    out = jnp.zeros((da0 + db0, da1 + db1), dtype=a.dtype)
    out = out.at[:da0, :da1].set(a)
    out = out.at[da0:, da1:].set(b)
    return out


def kernel(x, edge_attr, global_attr, params, edge_index):
    p1, p2 = params["w1"], params["w2"]
    s_idx = edge_index[0]
    r_idx = edge_index[1]
    u = global_attr
    u2 = jnp.concatenate([u, u], axis=1)

    WP = jnp.concatenate([p1["eb_W"][:16], p2["eb_W"][:16]], axis=1)
    cP = (jnp.concatenate([u @ p1["eb_W"][272:336], u @ p2["eb_W"][272:336]],
                          axis=1)
          + jnp.concatenate([p1["eb_b"], p2["eb_b"]])[None, :])
    onehot2 = jnp.zeros((D_NODE, 16), jnp.float32).at[2, 0].set(1.0)
    zeros16 = jnp.zeros((D_NODE, 16), jnp.float32)
    WR = jnp.concatenate([p1["eb_W"][16:144], p2["eb_W"][16:144], onehot2],
                         axis=1)
    WS = jnp.concatenate([p1["eb_W"][144:272], p2["eb_W"][144:272],
                          jnp.roll(onehot2, 1, axis=1)], axis=1)
    WN = jnp.concatenate([p1["nb_W"][64:192], p2["nb_W"][64:192]], axis=1)
    Wagg = _blockdiag(p1["nb_W"][:64], p2["nb_W"][:64])
    cn = (jnp.concatenate([u @ p1["nb_W"][192:256], u @ p2["nb_W"][192:256]],
                          axis=1)
          + jnp.concatenate([p1["nb_b"], p2["nb_b"]])[None, :])
    GWe = _blockdiag(p1["gb_W"][0:64], p2["gb_W"][0:64])
    GWn = _blockdiag(p1["gb_W"][64:128], p2["gb_W"][64:128])
    GWu = _blockdiag(p1["gb_W"][128:192], p2["gb_W"][128:192])
    gb = jnp.concatenate([p1["gb_b"], p2["gb_b"]])[None, :]
    Wbd = _blockdiag(p1["dec_W1"][0:64], p2["dec_W1"][0:64])
    Wdr = _blockdiag(p1["dec_W1"][64:128], p2["dec_W1"][64:128])
    Wds = _blockdiag(p1["dec_W1"][128:192], p2["dec_W1"][128:192])
    Wdu = _blockdiag(p1["dec_W1"][192:256], p2["dec_W1"][192:256])
    bd = jnp.concatenate([p1["dec_b1"], p2["dec_b1"]])[None, :]
    W2 = _blockdiag(p1["dec_W2"], p2["dec_W2"])
    b2 = jnp.concatenate([p1["dec_b2"], p2["dec_b2"]])[None, :]

    NB = 1000
    R_tab, S_tab, NX = pl.pallas_call(
        _tables_body,
        grid=(N // NB,),
        in_specs=[
            pl.BlockSpec((NB, D_NODE), lambda i: (i, 0)),
            pl.BlockSpec((D_NODE, 144), lambda i: (0, 0)),
            pl.BlockSpec((D_NODE, 144), lambda i: (0, 0)),
            pl.BlockSpec((D_NODE, 128), lambda i: (0, 0)),
        ],
        out_specs=[
            pl.BlockSpec((NB, 144), lambda i: (i, 0)),
            pl.BlockSpec((NB, 144), lambda i: (i, 0)),
            pl.BlockSpec((NB, 128), lambda i: (i, 0)),
        ],
        out_shape=[
            jax.ShapeDtypeStruct((N, 144), jnp.float32),
            jax.ShapeDtypeStruct((N, 144), jnp.float32),
            jax.ShapeDtypeStruct((N, 128), jnp.float32),
        ],
    )(x, WR, WS, WN)

    Rg = R_tab[r_idx]
    Sg = S_tab[s_idx]

    e1, xcols = pl.pallas_call(
        _edge1_body,
        grid=(E // BE,),
        in_specs=[
            pl.BlockSpec((BE, D_EDGE), lambda i: (i, 0)),
            pl.BlockSpec((BE, 144), lambda i: (i, 0)),
            pl.BlockSpec((BE, 144), lambda i: (i, 0)),
            pl.BlockSpec((D_EDGE, 128), lambda i: (0, 0)),
            pl.BlockSpec((1, 128), lambda i: (0, 0)),
        ],
        out_specs=[
            pl.BlockSpec((BE, 128), lambda i: (i, 0)),
            pl.BlockSpec((BE, 16), lambda i: (i, 0)),
        ],
        out_shape=[
            jax.ShapeDtypeStruct((E, 128), jnp.float32),
            jax.ShapeDtypeStruct((E, 16), jnp.float32),
        ],
    )(edge_attr, Rg, Sg, WP, cP)

    aggp = jax.ops.segment_sum(e1, r_idx, num_segments=N)[None]
    aggp = jnp.concatenate([aggp, jnp.zeros_like(aggp)], axis=0)

    Bd_r, Bd_s, cd = pl.pallas_call(
        _node_body,
        out_shape=[
            jax.ShapeDtypeStruct((N, 128), jnp.float32),
            jax.ShapeDtypeStruct((N, 128), jnp.float32),
            jax.ShapeDtypeStruct((1, 128), jnp.float32),
        ],
    )(aggp, NX, u2, Wagg, cn, GWe, GWn, GWu, gb, Wdr, Wds, Wdu, bd)

    Bgr = Bd_r[r_idx]
    Bgs = Bd_s[s_idx]

    ret = pl.pallas_call(
        _edge2_body,
        grid=(E // BE,),
        in_specs=[
            pl.BlockSpec((BE, 128), lambda i: (i, 0)),
            pl.BlockSpec((BE, 128), lambda i: (i, 0)),
            pl.BlockSpec((BE, 128), lambda i: (i, 0)),
            pl.BlockSpec((BE, 16), lambda i: (i, 0)),
            pl.BlockSpec((128, 128), lambda i: (0, 0)),
            pl.BlockSpec((1, 128), lambda i: (0, 0)),
            pl.BlockSpec((128, 2), lambda i: (0, 0)),
            pl.BlockSpec((1, 2), lambda i: (0, 0)),
        ],
        out_specs=pl.BlockSpec((BE, 1), lambda i: (i, 0)),
        out_shape=jax.ShapeDtypeStruct((E, 1), jnp.float32),
    )(e1, Bgr, Bgs, xcols, Wbd, cd, W2, b2)

    return ret


# speedup vs baseline: 2.7913x; 2.3513x over previous
import functools

import jax
import jax.numpy as jnp
from jax import lax
from jax.experimental import pallas as pl
from jax.experimental.pallas import tpu as pltpu
from jax.experimental.pallas import tpu_sc as plsc

N = 10000
E = 320000
D_NODE = 128
D_EDGE = 16
H = 64

BE = 2560

NC = 2
NS = 16
NW = NC * NS
EPW = E // NW
C = 80
NCH = EPW // C
AGG_TILES = 10
RPT = N // AGG_TILES
RB = 40


def _edge_p_body(edge_ref, wp_ref, cp_ref, p_ref):
    p_ref[...] = (jnp.dot(edge_ref[...], wp_ref[...],
                          preferred_element_type=jnp.float32) + cp_ref[...])


def _q_body(e1_ref, wbd_ref, cd_ref, q_ref):
    q_ref[...] = (jnp.dot(e1_ref[...], wbd_ref[...],
                          preferred_element_type=jnp.float32) + cd_ref[...])


def _final_body(h_ref, xr2_ref, xs2_ref, w2_ref, b2_ref, out_ref):
    o = jnp.dot(h_ref[...], w2_ref[...],
                preferred_element_type=jnp.float32) + b2_ref[...]
    out_ref[...] = o[:, 0:1] * (xr2_ref[...] - o[:, 1:2] * xs2_ref[...])


def _tables_body(x_ref, wr_ref, ws_ref, wn_ref, r_ref, s_ref, nx_ref):
    x = x_ref[...]
    r_ref[...] = jnp.dot(x, wr_ref[...], preferred_element_type=jnp.float32)
    s_ref[...] = jnp.dot(x, ws_ref[...], preferred_element_type=jnp.float32)
    nx_ref[...] = jnp.dot(x, wn_ref[...], preferred_element_type=jnp.float32)


def _node_body(aggp_ref, nx_ref, u_ref, wagg_ref, cn_ref, gw_e_ref, gw_n_ref,
               gw_u_ref, gb_ref, wdr_ref, wds_ref, wdu_ref, bd_ref,
               br_ref, bs_ref, cd_ref):
    agg = aggp_ref[0] + aggp_ref[1]
    n1 = jnp.maximum(
        jnp.dot(agg, wagg_ref[...], preferred_element_type=jnp.float32)
        + nx_ref[...] + cn_ref[...], 0.0)
    me = jnp.sum(agg, axis=0, keepdims=True) * (1.0 / E)
    mn = jnp.sum(n1, axis=0, keepdims=True) * (1.0 / N)
    u1 = jnp.maximum(
        jnp.dot(me, gw_e_ref[...], preferred_element_type=jnp.float32)
        + jnp.dot(mn, gw_n_ref[...], preferred_element_type=jnp.float32)
        + jnp.dot(u_ref[...], gw_u_ref[...], preferred_element_type=jnp.float32)
        + gb_ref[...], 0.0)
    br_ref[...] = jnp.dot(n1, wdr_ref[...], preferred_element_type=jnp.float32)
    bs_ref[...] = jnp.dot(n1, wds_ref[...], preferred_element_type=jnp.float32)
    cd_ref[...] = (jnp.dot(u1, wdu_ref[...], preferred_element_type=jnp.float32)
                   + bd_ref[...])


def _sc_edge1_body(r_hbm, s_hbm, rtab_hbm, stab_hbm, p_hbm,
                   e1_hbm, aggp_hbm,
                   idx_r_v, idx_s_v, rg_v, sg_v, p_v,
                   bounce_v, agg_sh, sem_r, sem_s):
    cid = lax.axis_index("c")
    sub = lax.axis_index("s")
    wid = sub * NC + cid

    zv = jnp.zeros((16,), jnp.float32)

    def _zero_row(j, _):
        for k in range(8):
            bounce_v[j, pl.ds(k * 16, 16)] = zv
        return 0

    lax.fori_loop(0, RB, _zero_row, 0)

    @pl.when(sub < AGG_TILES)
    def _init():
        for t in range(RPT // RB):
            pltpu.sync_copy(bounce_v, agg_sh.at[pl.ds(sub * RPT + t * RB, RB)])

    plsc.subcore_barrier()

    def _chunk(i, _):
        base = wid * EPW + i * C
        pltpu.sync_copy(r_hbm.at[pl.ds(base, C)], idx_r_v)
        pltpu.sync_copy(s_hbm.at[pl.ds(base, C)], idx_s_v)
        cr = pltpu.async_copy(rtab_hbm.at[idx_r_v], rg_v, sem_r)
        cs = pltpu.async_copy(stab_hbm.at[idx_s_v], sg_v, sem_s)
        pltpu.sync_copy(p_hbm.at[pl.ds(base, C)], p_v)
        cr.wait()
        cs.wait()

        def _row(j, _):
            for k in range(8):
                sl = pl.ds(k * 16, 16)
                pre = rg_v[j, sl] + sg_v[j, sl] + p_v[j, sl]
                p_v[j, sl] = jnp.maximum(pre, 0.0)
            return 0

        lax.fori_loop(0, C, _row, 0)
        pltpu.sync_copy(p_v, e1_hbm.at[pl.ds(base, C)])
        pltpu.sync_copy(p_v, agg_sh.at[idx_r_v], add=True)
        return 0

    lax.fori_loop(0, NCH, _chunk, 0)
    plsc.subcore_barrier()

    @pl.when(sub < AGG_TILES)
    def _readback():
        for t in range(RPT // RB):
            row0 = sub * RPT + t * RB
            pltpu.sync_copy(agg_sh.at[pl.ds(row0, RB)], bounce_v)
            pltpu.sync_copy(bounce_v, aggp_hbm.at[cid, pl.ds(row0, RB)])


def _sc_edge2_body(r_hbm, s_hbm, brtab_hbm, bstab_hbm, q_hbm,
                   h_hbm,
                   idx_r_v, idx_s_v, bgr_v, bgs_v, q_v, sem_r, sem_s):
    cid = lax.axis_index("c")
    sub = lax.axis_index("s")
    wid = sub * NC + cid

    def _chunk(i, _):
        base = wid * EPW + i * C
        pltpu.sync_copy(r_hbm.at[pl.ds(base, C)], idx_r_v)
        pltpu.sync_copy(s_hbm.at[pl.ds(base, C)], idx_s_v)
        cr = pltpu.async_copy(brtab_hbm.at[idx_r_v], bgr_v, sem_r)
        cs = pltpu.async_copy(bstab_hbm.at[idx_s_v], bgs_v, sem_s)
        pltpu.sync_copy(q_hbm.at[pl.ds(base, C)], q_v)
        cr.wait()
        cs.wait()

        def _row(j, _):
            for k in range(8):
                sl = pl.ds(k * 16, 16)
                pre = bgr_v[j, sl] + bgs_v[j, sl] + q_v[j, sl]
                q_v[j, sl] = jnp.maximum(pre, 0.0)
            return 0

        lax.fori_loop(0, C, _row, 0)
        pltpu.sync_copy(q_v, h_hbm.at[pl.ds(base, C)])
        return 0

    lax.fori_loop(0, NCH, _chunk, 0)


def _sc_x2_body(r_hbm, s_hbm, x2_hbm, xr2_hbm, xs2_hbm,
                idx_r_v, idx_s_v, x2_v, xr2_v, xs2_v):
    cid = lax.axis_index("c")
    sub = lax.axis_index("s")
    wid = sub * NC + cid
    base = wid * EPW

    pltpu.sync_copy(x2_hbm, x2_v)
    pltpu.sync_copy(r_hbm.at[pl.ds(base, EPW)], idx_r_v)
    pltpu.sync_copy(s_hbm.at[pl.ds(base, EPW)], idx_s_v)

    def _grp(k, _):
        sl16 = pl.ds(k * 16, 16)
        xr2_v[sl16] = plsc.load_gather(x2_v, [idx_r_v[sl16]])
        xs2_v[sl16] = plsc.load_gather(x2_v, [idx_s_v[sl16]])
        return 0

    lax.fori_loop(0, EPW // 16, _grp, 0)
    pltpu.sync_copy(xr2_v, xr2_hbm.at[pl.ds(base, EPW)])
    pltpu.sync_copy(xs2_v, xs2_hbm.at[pl.ds(base, EPW)])


_SC_MESH = plsc.VectorSubcoreMesh(core_axis_name="c", subcore_axis_name="s")

_sc_edge1 = functools.partial(
    pl.kernel,
    out_type=[
        jax.ShapeDtypeStruct((E, 128), jnp.float32),
        jax.ShapeDtypeStruct((NC, N, 128), jnp.float32),
    ],
    mesh=_SC_MESH,
    scratch_types=[
        pltpu.VMEM((C,), jnp.int32),
        pltpu.VMEM((C,), jnp.int32),
        pltpu.VMEM((C, 128), jnp.float32),
        pltpu.VMEM((C, 128), jnp.float32),
        pltpu.VMEM((C, 128), jnp.float32),
        pltpu.VMEM((RB, 128), jnp.float32),
        pltpu.VMEM_SHARED((N, 128), jnp.float32),
        pltpu.SemaphoreType.DMA,
        pltpu.SemaphoreType.DMA,
    ],
)(_sc_edge1_body)

_sc_x2 = functools.partial(
    pl.kernel,
    out_type=[
        jax.ShapeDtypeStruct((E,), jnp.float32),
        jax.ShapeDtypeStruct((E,), jnp.float32),
    ],
    mesh=_SC_MESH,
    scratch_types=[
        pltpu.VMEM((EPW,), jnp.int32),
        pltpu.VMEM((EPW,), jnp.int32),
        pltpu.VMEM((N,), jnp.float32),
        pltpu.VMEM((EPW,), jnp.float32),
        pltpu.VMEM((EPW,), jnp.float32),
    ],
    compiler_params=pltpu.CompilerParams(needs_layout_passes=False),
)(_sc_x2_body)

_sc_edge2 = functools.partial(
    pl.kernel,
    out_type=jax.ShapeDtypeStruct((E, 128), jnp.float32),
    mesh=_SC_MESH,
    scratch_types=[
        pltpu.VMEM((C,), jnp.int32),
        pltpu.VMEM((C,), jnp.int32),
        pltpu.VMEM((C, 128), jnp.float32),
        pltpu.VMEM((C, 128), jnp.float32),
        pltpu.VMEM((C, 128), jnp.float32),
        pltpu.SemaphoreType.DMA,
        pltpu.SemaphoreType.DMA,
    ],
)(_sc_edge2_body)


def _blockdiag(a, b):
    da0, da1 = a.shape
    db0, db1 = b.shape
    out = jnp.zeros((da0 + db0, da1 + db1), dtype=a.dtype)
    out = out.at[:da0, :da1].set(a)
    out = out.at[da0:, da1:].set(b)
    return out


def kernel(x, edge_attr, global_attr, params, edge_index):
    p1, p2 = params["w1"], params["w2"]
    s_idx = edge_index[0]
    r_idx = edge_index[1]
    u = global_attr
    u2 = jnp.concatenate([u, u], axis=1)

    WP = jnp.concatenate([p1["eb_W"][:16], p2["eb_W"][:16]], axis=1)
    cP = (jnp.concatenate([u @ p1["eb_W"][272:336], u @ p2["eb_W"][272:336]],
                          axis=1)
          + jnp.concatenate([p1["eb_b"], p2["eb_b"]])[None, :])
    WR = jnp.concatenate([p1["eb_W"][16:144], p2["eb_W"][16:144]],
                         axis=1)
    WS = jnp.concatenate([p1["eb_W"][144:272], p2["eb_W"][144:272]],
                         axis=1)
    WN = jnp.concatenate([p1["nb_W"][64:192], p2["nb_W"][64:192]], axis=1)
    Wagg = _blockdiag(p1["nb_W"][:64], p2["nb_W"][:64])
    cn = (jnp.concatenate([u @ p1["nb_W"][192:256], u @ p2["nb_W"][192:256]],
                          axis=1)
          + jnp.concatenate([p1["nb_b"], p2["nb_b"]])[None, :])
    GWe = _blockdiag(p1["gb_W"][0:64], p2["gb_W"][0:64])
    GWn = _blockdiag(p1["gb_W"][64:128], p2["gb_W"][64:128])
    GWu = _blockdiag(p1["gb_W"][128:192], p2["gb_W"][128:192])
    gb = jnp.concatenate([p1["gb_b"], p2["gb_b"]])[None, :]
    Wbd = _blockdiag(p1["dec_W1"][0:64], p2["dec_W1"][0:64])
    Wdr = _blockdiag(p1["dec_W1"][64:128], p2["dec_W1"][64:128])
    Wds = _blockdiag(p1["dec_W1"][128:192], p2["dec_W1"][128:192])
    Wdu = _blockdiag(p1["dec_W1"][192:256], p2["dec_W1"][192:256])
    bd = jnp.concatenate([p1["dec_b1"], p2["dec_b1"]])[None, :]
    W2 = _blockdiag(p1["dec_W2"], p2["dec_W2"])
    b2 = jnp.concatenate([p1["dec_b2"], p2["dec_b2"]])[None, :]

    NB = 1000
    R_tab, S_tab, NX = pl.pallas_call(
        _tables_body,
        grid=(N // NB,),
        in_specs=[
            pl.BlockSpec((NB, D_NODE), lambda i: (i, 0)),
            pl.BlockSpec((D_NODE, 128), lambda i: (0, 0)),
            pl.BlockSpec((D_NODE, 128), lambda i: (0, 0)),
            pl.BlockSpec((D_NODE, 128), lambda i: (0, 0)),
        ],
        out_specs=[
            pl.BlockSpec((NB, 128), lambda i: (i, 0)),
            pl.BlockSpec((NB, 128), lambda i: (i, 0)),
            pl.BlockSpec((NB, 128), lambda i: (i, 0)),
        ],
        out_shape=[
            jax.ShapeDtypeStruct((N, 128), jnp.float32),
            jax.ShapeDtypeStruct((N, 128), jnp.float32),
            jax.ShapeDtypeStruct((N, 128), jnp.float32),
        ],
    )(x, WR, WS, WN)

    P = pl.pallas_call(
        _edge_p_body,
        grid=(E // BE,),
        in_specs=[
            pl.BlockSpec((BE, D_EDGE), lambda i: (i, 0)),
            pl.BlockSpec((D_EDGE, 128), lambda i: (0, 0)),
            pl.BlockSpec((1, 128), lambda i: (0, 0)),
        ],
        out_specs=pl.BlockSpec((BE, 128), lambda i: (i, 0)),
        out_shape=jax.ShapeDtypeStruct((E, 128), jnp.float32),
    )(edge_attr, WP, cP)

    x2 = x[:, 2]
    xr2, xs2 = _sc_x2(r_idx, s_idx, x2)

    e1, aggp = _sc_edge1(r_idx, s_idx, R_tab, S_tab, P)

    Bd_r, Bd_s, cd = pl.pallas_call(
        _node_body,
        out_shape=[
            jax.ShapeDtypeStruct((N, 128), jnp.float32),
            jax.ShapeDtypeStruct((N, 128), jnp.float32),
            jax.ShapeDtypeStruct((1, 128), jnp.float32),
        ],
    )(aggp, NX, u2, Wagg, cn, GWe, GWn, GWu, gb, Wdr, Wds, Wdu, bd)

    Q = pl.pallas_call(
        _q_body,
        grid=(E // BE,),
        in_specs=[
            pl.BlockSpec((BE, 128), lambda i: (i, 0)),
            pl.BlockSpec((128, 128), lambda i: (0, 0)),
            pl.BlockSpec((1, 128), lambda i: (0, 0)),
        ],
        out_specs=pl.BlockSpec((BE, 128), lambda i: (i, 0)),
        out_shape=jax.ShapeDtypeStruct((E, 128), jnp.float32),
    )(e1, Wbd, cd)

    h = _sc_edge2(r_idx, s_idx, Bd_r, Bd_s, Q)

    ret = pl.pallas_call(
        _final_body,
        grid=(E // BE,),
        in_specs=[
            pl.BlockSpec((BE, 128), lambda i: (i, 0)),
            pl.BlockSpec((BE, 1), lambda i: (i, 0)),
            pl.BlockSpec((BE, 1), lambda i: (i, 0)),
            pl.BlockSpec((128, 2), lambda i: (0, 0)),
            pl.BlockSpec((1, 2), lambda i: (0, 0)),
        ],
        out_specs=pl.BlockSpec((BE, 1), lambda i: (i, 0)),
        out_shape=jax.ShapeDtypeStruct((E, 1), jnp.float32),
    )(h, xr2.reshape(E, 1), xs2.reshape(E, 1), W2, b2)

    return ret
